# baseline (speedup 1.0000x reference)
import jax
import jax.numpy as jnp
from jax import lax
from jax.experimental import pallas as pl
from jax.experimental.pallas import tpu as pltpu
from jax.experimental.pallas import tpu_sc as plsc

_N_NODES = 10000
_N_EDGES = 320000
_D = 128

_NC = 2
_NS = 16
_NW = _NC * _NS
_L = 16

_EPW = _N_EDGES // _NW
_CH = 80
_NCHUNK = _EPW // _CH
_NGRP = _CH // _L


def _sc_kernel(e1_hbm, e2_hbm, ei_hbm, out_hbm,
               tab_sh, sp0, dp0, si0, di0, sp1, dp1, si1, di1,
               sr0, dr0, sr1, dr1, outc0, outc1,
               sem0, sem1, semh0, semh1, semi0, semi1, semo0, semo1):
    cid = lax.axis_index("c")
    sid = lax.axis_index("s")
    wid = sid * _NC + cid
    base = wid * _EPW

    @pl.when(sid == 0)
    def _():
        pltpu.sync_copy(e1_hbm, tab_sh)

    plsc.subcore_barrier()

    def fire_idx(c, sp, dp, semi):
        pltpu.async_copy(ei_hbm.at[pl.ds(base + c * _CH, _CH)], sp, semi)
        pltpu.async_copy(ei_hbm.at[pl.ds(_N_EDGES + base + c * _CH, _CH)], dp, semi)

    def wait_idx(sp, dp, semi):
        pltpu.make_async_copy(ei_hbm.at[pl.ds(0, _CH)], sp, semi).wait()
        pltpu.make_async_copy(ei_hbm.at[pl.ds(0, _CH)], dp, semi).wait()

    def adopt_idx(sp, dp, si, di):
        for i in range(_CH // _L):
            si[pl.ds(i * _L, _L)] = sp[pl.ds(i * _L, _L)]
            di[pl.ds(i * _L, _L)] = dp[pl.ds(i * _L, _L)]

    def fire_gathers(si, di, sr, dr, sem, semh):
        pltpu.async_copy(tab_sh.at[si], sr, sem)
        pltpu.async_copy(e2_hbm.at[di], dr, semh)

    def wait_gathers(si, di, sr, dr, sem, semh):
        pltpu.make_async_copy(tab_sh.at[si], sr, sem).wait()
        pltpu.make_async_copy(e2_hbm.at[di], dr, semh).wait()

    lane_iota = lax.broadcasted_iota(jnp.int32, (_L,), 0)

    def compute(sr, dr, outc):
        def grp_body(g, _):
            e0 = g * _L
            chains = [jnp.zeros((_L,), jnp.float32) for _ in range(4)]
            for e in range(_L):
                row = e0 + e
                prods = [sr[row, pl.ds(j * _L, _L)] * dr[row, pl.ds(j * _L, _L)]
                         for j in range(8)]
                s4 = [prods[k] + prods[k + 4] for k in range(4)]
                p = (s4[0] + s4[2]) + (s4[1] + s4[3])
                tot = jnp.sum(p)
                chains[e % 4] = jnp.where(lane_iota == e, tot, chains[e % 4])
            vec = (chains[0] + chains[1]) + (chains[2] + chains[3])
            outc[pl.ds(e0, _L)] = vec
            return 0

        lax.fori_loop(0, _NGRP, grp_body, 0)

    def step(c, sp, dp, si, di, sr, dr, outc, sem, semh, semi, semo):
        @pl.when(c >= 2)
        def _():
            pltpu.make_async_copy(outc, out_hbm.at[pl.ds(0, _CH)], semo).wait()

        wait_gathers(si, di, sr, dr, sem, semh)
        compute(sr, dr, outc)

        @pl.when(c + 2 < _NCHUNK)
        def _():
            wait_idx(sp, dp, semi)
            adopt_idx(sp, dp, si, di)
            fire_gathers(si, di, sr, dr, sem, semh)

            @pl.when(c + 4 < _NCHUNK)
            def _():
                fire_idx(c + 4, sp, dp, semi)

        pltpu.async_copy(outc, out_hbm.at[pl.ds(base + c * _CH, _CH)], semo)

    pltpu.sync_copy(ei_hbm.at[pl.ds(base, _CH)], si0)
    pltpu.sync_copy(ei_hbm.at[pl.ds(_N_EDGES + base, _CH)], di0)
    fire_gathers(si0, di0, sr0, dr0, sem0, semh0)
    pltpu.sync_copy(ei_hbm.at[pl.ds(base + _CH, _CH)], si1)
    pltpu.sync_copy(ei_hbm.at[pl.ds(_N_EDGES + base + _CH, _CH)], di1)
    fire_gathers(si1, di1, sr1, dr1, sem1, semh1)
    fire_idx(2, sp0, dp0, semi0)
    fire_idx(3, sp1, dp1, semi1)

    def chunk_body(c, _):
        @pl.when(c % 2 == 0)
        def _():
            step(c, sp0, dp0, si0, di0, sr0, dr0, outc0, sem0, semh0, semi0, semo0)

        @pl.when(c % 2 == 1)
        def _():
            step(c, sp1, dp1, si1, di1, sr1, dr1, outc1, sem1, semh1, semi1, semo1)

        return 0

    lax.fori_loop(0, _NCHUNK, chunk_body, 0)
    pltpu.make_async_copy(outc0, out_hbm.at[pl.ds(0, _CH)], semo0).wait()
    pltpu.make_async_copy(outc1, out_hbm.at[pl.ds(0, _CH)], semo1).wait()


@jax.jit
def _run(e1, e2, ei):
    mesh = plsc.VectorSubcoreMesh(core_axis_name="c", subcore_axis_name="s")
    return pl.kernel(
        _sc_kernel,
        out_type=jax.ShapeDtypeStruct((_N_EDGES,), jnp.float32),
        mesh=mesh,
        compiler_params=pltpu.CompilerParams(needs_layout_passes=False),
        scratch_types=[
            pltpu.VMEM_SHARED((_N_NODES, _D), jnp.float32),
            pltpu.VMEM((_CH,), jnp.int32),
            pltpu.VMEM((_CH,), jnp.int32),
            pltpu.VMEM((_CH,), jnp.int32),
            pltpu.VMEM((_CH,), jnp.int32),
            pltpu.VMEM((_CH,), jnp.int32),
            pltpu.VMEM((_CH,), jnp.int32),
            pltpu.VMEM((_CH,), jnp.int32),
            pltpu.VMEM((_CH,), jnp.int32),
            pltpu.VMEM((_CH, _D), jnp.float32),
            pltpu.VMEM((_CH, _D), jnp.float32),
            pltpu.VMEM((_CH, _D), jnp.float32),
            pltpu.VMEM((_CH, _D), jnp.float32),
            pltpu.VMEM((_CH,), jnp.float32),
            pltpu.VMEM((_CH,), jnp.float32),
            pltpu.SemaphoreType.DMA,
            pltpu.SemaphoreType.DMA,
            pltpu.SemaphoreType.DMA,
            pltpu.SemaphoreType.DMA,
            pltpu.SemaphoreType.DMA,
            pltpu.SemaphoreType.DMA,
            pltpu.SemaphoreType.DMA,
            pltpu.SemaphoreType.DMA,
        ],
    )(e1, e2, ei)


def kernel(embedding_1, embedding_2, edge_label_index):
    ei = edge_label_index.astype(jnp.int32).reshape(2 * _N_EDGES)
    return _run(embedding_1, embedding_2, ei)


# speedup vs baseline: 1.7289x; 1.7289x over previous
import jax
import jax.numpy as jnp
from jax import lax
from jax.experimental import pallas as pl
from jax.experimental.pallas import tpu as pltpu
from jax.experimental.pallas import tpu_sc as plsc

_N_NODES = 10000
_N_EDGES = 320000
_D = 128
_DW = _D // 2

_NC = 2
_NS = 16
_NW = _NC * _NS
_L = 16

_EPW = _N_EDGES // _NW
_CH = 80
_NCHUNK = _EPW // _CH
_NGRP = _CH // _L

_BR = 8
_NBLK = _N_NODES // _BR
_BPT = -(-_NBLK // _NS)


def _sc_kernel(e1_hbm, e2_hbm, ei_hbm, out_hbm,
               tab_sh, t1a, t2a, t1b, t2b, cb,
               sp0, dp0, si0, di0, sp1, dp1, si1, di1,
               sr0, dr0, sr1, dr1, outc0, outc1,
               semca, semcb, sem0, sem1, semi0, semi1, semo0, semo1):
    cid = lax.axis_index("c")
    sid = lax.axis_index("s")
    wid = sid * _NC + cid
    base = wid * _EPW

    def fire_block_reads(b, t1, t2, semc):
        pltpu.async_copy(e1_hbm.at[pl.ds(b * _BR, _BR)], t1, semc)
        pltpu.async_copy(e2_hbm.at[pl.ds(b * _BR, _BR)], t2, semc)

    def wait_block_reads(t1, t2, semc):
        pltpu.make_async_copy(e1_hbm.at[pl.ds(0, _BR)], t1, semc).wait()
        pltpu.make_async_copy(e2_hbm.at[pl.ds(0, _BR)], t2, semc).wait()

    def convert_block(b, t1, t2):
        for r in range(_BR):
            for j in range(4):
                u = t1[r, pl.ds(j * 2 * _L, _L)]
                v = t1[r, pl.ds(j * 2 * _L + _L, _L)]
                cb[r, pl.ds(j * _L, _L)] = plsc.bitcast(
                    plsc.pack(u, v, format=plsc.PackFormat.INTERLEAVED),
                    jnp.int32)
            for j in range(4):
                u = t2[r, pl.ds(j * 2 * _L, _L)]
                v = t2[r, pl.ds(j * 2 * _L + _L, _L)]
                cb[r, pl.ds(_DW + j * _L, _L)] = plsc.bitcast(
                    plsc.pack(u, v, format=plsc.PackFormat.INTERLEAVED),
                    jnp.int32)
        pltpu.sync_copy(cb, tab_sh.at[pl.ds(b * _BR, _BR)])

    fire_block_reads(sid, t1a, t2a, semca)

    def conv_step(t, bufs, semc, obufs, osemc):
        b = sid + _NS * t

        @pl.when(b < _NBLK)
        def _():
            wait_block_reads(bufs[0], bufs[1], semc)
            nb = b + _NS

            @pl.when(nb < _NBLK)
            def _():
                fire_block_reads(nb, obufs[0], obufs[1], osemc)

            convert_block(b, bufs[0], bufs[1])

    def conv_pair(tp, _):
        conv_step(2 * tp, (t1a, t2a), semca, (t1b, t2b), semcb)
        conv_step(2 * tp + 1, (t1b, t2b), semcb, (t1a, t2a), semca)
        return 0

    lax.fori_loop(0, _BPT // 2, conv_pair, 0)
    if _BPT % 2:
        conv_step(_BPT - 1, (t1a, t2a), semca, (t1b, t2b), semcb)

    plsc.subcore_barrier()

    def fire_idx(c, sp, dp, semi):
        pltpu.async_copy(ei_hbm.at[pl.ds(base + c * _CH, _CH)], sp, semi)
        pltpu.async_copy(ei_hbm.at[pl.ds(_N_EDGES + base + c * _CH, _CH)],
                         dp, semi)

    def wait_idx(sp, dp, semi):
        pltpu.make_async_copy(ei_hbm.at[pl.ds(0, _CH)], sp, semi).wait()
        pltpu.make_async_copy(ei_hbm.at[pl.ds(0, _CH)], dp, semi).wait()

    def adopt_idx(sp, dp, si, di):
        for i in range(_CH // _L):
            si[pl.ds(i * _L, _L)] = sp[pl.ds(i * _L, _L)]
            di[pl.ds(i * _L, _L)] = dp[pl.ds(i * _L, _L)]

    def fire_gathers(si, di, sr, dr, sem):
        pltpu.async_copy(tab_sh.at[si], sr, sem)
        pltpu.async_copy(tab_sh.at[di], dr, sem)

    def wait_gathers(si, di, sr, dr, sem):
        pltpu.make_async_copy(tab_sh.at[si], sr, sem).wait()
        pltpu.make_async_copy(tab_sh.at[di], dr, sem).wait()

    lane_iota = lax.broadcasted_iota(jnp.int32, (_L,), 0)

    def compute(sr, dr, outc):
        def grp_body(g, _):
            e0 = g * _L
            chains = [jnp.zeros((_L,), jnp.float32) for _ in range(4)]
            for e in range(_L):
                row = e0 + e
                prods = []
                for j in range(4):
                    sw = plsc.bitcast(sr[row, pl.ds(j * _L, _L)], jnp.bfloat16)
                    dw = plsc.bitcast(dr[row, pl.ds(_DW + j * _L, _L)],
                                      jnp.bfloat16)
                    sa, sb = plsc.unpack(sw, format=plsc.PackFormat.INTERLEAVED,
                                         preferred_element_type=jnp.float32)
                    da, db = plsc.unpack(dw, format=plsc.PackFormat.INTERLEAVED,
                                         preferred_element_type=jnp.float32)
                    prods.append(sa * da)
                    prods.append(sb * db)
                s4 = [prods[k] + prods[k + 4] for k in range(4)]
                p = (s4[0] + s4[2]) + (s4[1] + s4[3])
                tot = jnp.sum(p)
                chains[e % 4] = jnp.where(lane_iota == e, tot, chains[e % 4])
            vec = (chains[0] + chains[1]) + (chains[2] + chains[3])
            outc[pl.ds(e0, _L)] = vec
            return 0

        lax.fori_loop(0, _NGRP, grp_body, 0)

    def step(c, sp, dp, si, di, sr, dr, outc, sem, semi, semo):
        @pl.when(c >= 2)
        def _():
            pltpu.make_async_copy(outc, out_hbm.at[pl.ds(0, _CH)], semo).wait()

        wait_gathers(si, di, sr, dr, sem)
        compute(sr, dr, outc)

        @pl.when(c + 2 < _NCHUNK)
        def _():
            wait_idx(sp, dp, semi)
            adopt_idx(sp, dp, si, di)
            fire_gathers(si, di, sr, dr, sem)

            @pl.when(c + 4 < _NCHUNK)
            def _():
                fire_idx(c + 4, sp, dp, semi)

        pltpu.async_copy(outc, out_hbm.at[pl.ds(base + c * _CH, _CH)], semo)

    pltpu.sync_copy(ei_hbm.at[pl.ds(base, _CH)], si0)
    pltpu.sync_copy(ei_hbm.at[pl.ds(_N_EDGES + base, _CH)], di0)
    fire_gathers(si0, di0, sr0, dr0, sem0)
    pltpu.sync_copy(ei_hbm.at[pl.ds(base + _CH, _CH)], si1)
    pltpu.sync_copy(ei_hbm.at[pl.ds(_N_EDGES + base + _CH, _CH)], di1)
    fire_gathers(si1, di1, sr1, dr1, sem1)
    fire_idx(2, sp0, dp0, semi0)
    fire_idx(3, sp1, dp1, semi1)

    def chunk_body(c, _):
        @pl.when(c % 2 == 0)
        def _():
            step(c, sp0, dp0, si0, di0, sr0, dr0, outc0, sem0, semi0, semo0)

        @pl.when(c % 2 == 1)
        def _():
            step(c, sp1, dp1, si1, di1, sr1, dr1, outc1, sem1, semi1, semo1)

        return 0

    lax.fori_loop(0, _NCHUNK, chunk_body, 0)
    pltpu.make_async_copy(outc0, out_hbm.at[pl.ds(0, _CH)], semo0).wait()
    pltpu.make_async_copy(outc1, out_hbm.at[pl.ds(0, _CH)], semo1).wait()


@jax.jit
def _run(e1, e2, ei):
    mesh = plsc.VectorSubcoreMesh(core_axis_name="c", subcore_axis_name="s")
    return pl.kernel(
        _sc_kernel,
        out_type=jax.ShapeDtypeStruct((_N_EDGES,), jnp.float32),
        mesh=mesh,
        compiler_params=pltpu.CompilerParams(needs_layout_passes=False),
        scratch_types=[
            pltpu.VMEM_SHARED((_N_NODES, _D), jnp.int32),
            pltpu.VMEM((_BR, _D), jnp.float32),
            pltpu.VMEM((_BR, _D), jnp.float32),
            pltpu.VMEM((_BR, _D), jnp.float32),
            pltpu.VMEM((_BR, _D), jnp.float32),
            pltpu.VMEM((_BR, _D), jnp.int32),
            pltpu.VMEM((_CH,), jnp.int32),
            pltpu.VMEM((_CH,), jnp.int32),
            pltpu.VMEM((_CH,), jnp.int32),
            pltpu.VMEM((_CH,), jnp.int32),
            pltpu.VMEM((_CH,), jnp.int32),
            pltpu.VMEM((_CH,), jnp.int32),
            pltpu.VMEM((_CH,), jnp.int32),
            pltpu.VMEM((_CH,), jnp.int32),
            pltpu.VMEM((_CH, _D), jnp.int32),
            pltpu.VMEM((_CH, _D), jnp.int32),
            pltpu.VMEM((_CH, _D), jnp.int32),
            pltpu.VMEM((_CH, _D), jnp.int32),
            pltpu.VMEM((_CH,), jnp.float32),
            pltpu.VMEM((_CH,), jnp.float32),
            pltpu.SemaphoreType.DMA,
            pltpu.SemaphoreType.DMA,
            pltpu.SemaphoreType.DMA,
            pltpu.SemaphoreType.DMA,
            pltpu.SemaphoreType.DMA,
            pltpu.SemaphoreType.DMA,
            pltpu.SemaphoreType.DMA,
            pltpu.SemaphoreType.DMA,
        ],
    )(e1, e2, ei)


def kernel(embedding_1, embedding_2, edge_label_index):
    ei = edge_label_index.astype(jnp.int32).reshape(2 * _N_EDGES)
    return _run(embedding_1, embedding_2, ei)
